# baseline (speedup 1.0000x reference)
import functools

import jax
import jax.numpy as jnp
from jax import lax
from jax.experimental import pallas as pl
from jax.experimental.pallas import tpu as pltpu
from jax.experimental.pallas import tpu_sc as plsc

_N = 1_000_000
_C = 100
_NBINS = 15


_R = 8192
_NB = (_N + _R - 1) // _R


def _tc_body(p_ref, conf_ref):
    conf_ref[...] = jnp.max(p_ref[...], axis=1)


def _tc_call(preds):
    return pl.pallas_call(
        _tc_body,
        grid=(_NB,),
        in_specs=[pl.BlockSpec((_R, _C), lambda i: (i, 0))],
        out_specs=[pl.BlockSpec((_R,), lambda i: (i,))],
        out_shape=[jax.ShapeDtypeStruct((_N,), jnp.float32)],
        compiler_params=pltpu.CompilerParams(
            dimension_semantics=("arbitrary",),
        ),
    )(preds)[0]


_NW = 32
_PW = 31248
_PLAST = _N - (_NW - 1) * _PW
_NIT = _PW // 16
_NIT_LAST = _PLAST // 16


def _sc_hist(conf_hbm, pflat_hbm, lab_hbm, out_hbm,
             conf_v, lab_v, idx_v, pick_v, acc_n, acc_r, acc_c, fold_v, sem):
    wid = lax.axis_index("s") * 2 + lax.axis_index("c")
    is_last = wid == _NW - 1
    base = wid * _PW

    zeros16 = jnp.zeros((16,), jnp.float32)
    for j in range(16):
        acc_n[pl.ds(j * 16, 16)] = zeros16
        acc_r[pl.ds(j * 16, 16)] = zeros16
        acc_c[pl.ds(j * 16, 16)] = zeros16

    @pl.when(is_last)
    def _():
        pltpu.sync_copy(conf_hbm.at[pl.ds(base, _PLAST)], conf_v)
        pltpu.sync_copy(lab_hbm.at[pl.ds(base, _PLAST)], lab_v)

    @pl.when(jnp.logical_not(is_last))
    def _():
        pltpu.sync_copy(conf_hbm.at[pl.ds(base, _PW)], conf_v.at[pl.ds(0, _PW)])
        pltpu.sync_copy(lab_hbm.at[pl.ds(base, _PW)], lab_v.at[pl.ds(0, _PW)])
        for j in range((_PLAST - _PW) // 16):
            idx_v[pl.ds(_PW + j * 16, 16)] = jnp.zeros((16,), jnp.int32)

    lane = lax.iota(jnp.int32, 16)
    niter = jnp.where(is_last, _NIT_LAST, _NIT)

    rowbase = (base + lane) * _C

    def idx_body(i, _):
        off = i * 16
        l = lab_v[pl.ds(off, 16)]
        idx_v[pl.ds(off, 16)] = rowbase + off * _C + l
        return _

    lax.fori_loop(0, niter, idx_body, 0)

    pltpu.async_copy(pflat_hbm.at[idx_v], pick_v, sem).wait()

    lane16 = lane * 16
    ones = jnp.ones((16,), jnp.float32)

    def body(i, _):
        off = i * 16
        c = conf_v[pl.ds(off, 16)]
        p = pick_v[pl.ds(off, 16)]
        r = jnp.where(p == c, 1.0, 0.0).astype(jnp.float32)
        k = jnp.minimum((c * jnp.float32(_NBINS)).astype(jnp.int32), _NBINS - 1)
        k = jnp.where(c > 0.0, k, jnp.int32(15))
        idx = lane16 + k
        plsc.store_scatter(acc_n, [idx], plsc.load_gather(acc_n, [idx]) + ones)
        plsc.store_scatter(acc_r, [idx], plsc.load_gather(acc_r, [idx]) + r)
        plsc.store_scatter(acc_c, [idx], plsc.load_gather(acc_c, [idx]) + c)
        return _

    lax.fori_loop(0, niter, body, 0)

    def _fold_and_store(acc, slot):
        s = acc[pl.ds(0, 16)]
        for j in range(1, 16):
            s = s + acc[pl.ds(j * 16, 16)]
        fold_v[...] = s
        pltpu.sync_copy(fold_v, out_hbm.at[pl.ds(wid * 48 + slot * 16, 16)])

    _fold_and_store(acc_n, 0)
    _fold_and_store(acc_r, 1)
    _fold_and_store(acc_c, 2)


@functools.cache
def _sc_call():
    mesh = plsc.VectorSubcoreMesh(core_axis_name="c", subcore_axis_name="s")
    return pl.kernel(
        _sc_hist,
        mesh=mesh,
        out_type=jax.ShapeDtypeStruct((_NW * 3 * 16,), jnp.float32),
        scratch_types=[
            pltpu.VMEM((_PLAST,), jnp.float32),
            pltpu.VMEM((_PLAST,), jnp.int32),
            pltpu.VMEM((_PLAST,), jnp.int32),
            pltpu.VMEM((_PLAST,), jnp.float32),
            pltpu.VMEM((256,), jnp.float32),
            pltpu.VMEM((256,), jnp.float32),
            pltpu.VMEM((256,), jnp.float32),
            pltpu.VMEM((16,), jnp.float32),
            pltpu.SemaphoreType.DMA,
        ],
        compiler_params=pltpu.CompilerParams(needs_layout_passes=False),
    )


def kernel(preds, labels):
    labels = labels.astype(jnp.int32)
    conf = _tc_call(preds)
    pflat = preds.reshape(-1)
    parts = _sc_call()(conf, pflat, labels).reshape(_NW, 3, 16)
    tot = jnp.sum(parts, axis=0)
    cnt = tot[0, :_NBINS]
    cor = tot[1, :_NBINS]
    cnf = tot[2, :_NBINS]
    n = jnp.float32(_N)
    safe = jnp.maximum(cnt, 1.0)
    terms = jnp.abs(cnf / safe - cor / safe) * (cnt / n)
    ece = jnp.sum(jnp.where(cnt > 0, terms, 0.0))
    return ece.astype(jnp.float32)


# speedup vs baseline: 1.0930x; 1.0930x over previous
import functools

import jax
import jax.numpy as jnp
from jax import lax
from jax.experimental import pallas as pl
from jax.experimental.pallas import tpu as pltpu
from jax.experimental.pallas import tpu_sc as plsc

_N = 1_000_000
_C = 100
_NBINS = 15


_R = 8192
_NB = (_N + _R - 1) // _R


def _tc_body(p_ref, conf_ref):
    x = p_ref[...]
    cmax = jnp.max(x, axis=1, keepdims=True)
    conf_ref[...] = jnp.swapaxes(cmax, 0, 1)[None]


def _tc_call(preds):
    out2 = pl.pallas_call(
        _tc_body,
        grid=(_NB,),
        in_specs=[pl.BlockSpec((_R, _C), lambda i: (i, 0))],
        out_specs=[pl.BlockSpec((1, 1, _R), lambda i: (i, 0, 0))],
        out_shape=[jax.ShapeDtypeStruct((_NB, 1, _R), jnp.float32)],
        compiler_params=pltpu.CompilerParams(
            dimension_semantics=("arbitrary",),
        ),
    )(preds)[0]
    return out2.reshape(-1)


_NW = 32
_PW = 31248
_PLAST = _N - (_NW - 1) * _PW
_NIT = _PW // 16
_NIT_LAST = _PLAST // 16


def _sc_hist(conf_hbm, pflat_hbm, lab_hbm, out_hbm,
             conf_v, lab_v, idx_v, pick_v, acc_n, acc_r, acc_c, fold_v, sem):
    wid = lax.axis_index("s") * 2 + lax.axis_index("c")
    is_last = wid == _NW - 1
    base = wid * _PW

    zeros16 = jnp.zeros((16,), jnp.float32)
    for j in range(16):
        acc_n[pl.ds(j * 16, 16)] = zeros16
        acc_r[pl.ds(j * 16, 16)] = zeros16
        acc_c[pl.ds(j * 16, 16)] = zeros16

    @pl.when(is_last)
    def _():
        pltpu.sync_copy(conf_hbm.at[pl.ds(base, _PLAST)], conf_v)
        pltpu.sync_copy(lab_hbm.at[pl.ds(base, _PLAST)], lab_v)

    @pl.when(jnp.logical_not(is_last))
    def _():
        pltpu.sync_copy(conf_hbm.at[pl.ds(base, _PW)], conf_v.at[pl.ds(0, _PW)])
        pltpu.sync_copy(lab_hbm.at[pl.ds(base, _PW)], lab_v.at[pl.ds(0, _PW)])
        for j in range((_PLAST - _PW) // 16):
            idx_v[pl.ds(_PW + j * 16, 16)] = jnp.zeros((16,), jnp.int32)

    lane = lax.iota(jnp.int32, 16)
    niter = jnp.where(is_last, _NIT_LAST, _NIT)

    rowbase = (base + lane) * _C

    def idx_body(i, _):
        off = i * 16
        l = lab_v[pl.ds(off, 16)]
        idx_v[pl.ds(off, 16)] = rowbase + off * _C + l
        return _

    lax.fori_loop(0, niter, idx_body, 0)

    pltpu.async_copy(pflat_hbm.at[idx_v], pick_v, sem).wait()

    lane16 = lane * 16
    ones = jnp.ones((16,), jnp.float32)

    def body(i, _):
        off = i * 16
        c = conf_v[pl.ds(off, 16)]
        p = pick_v[pl.ds(off, 16)]
        r = jnp.where(p == c, 1.0, 0.0).astype(jnp.float32)
        k = jnp.minimum((c * jnp.float32(_NBINS)).astype(jnp.int32), _NBINS - 1)
        k = jnp.where(c > 0.0, k, jnp.int32(15))
        idx = lane16 + k
        plsc.store_scatter(acc_n, [idx], plsc.load_gather(acc_n, [idx]) + ones)
        plsc.store_scatter(acc_r, [idx], plsc.load_gather(acc_r, [idx]) + r)
        plsc.store_scatter(acc_c, [idx], plsc.load_gather(acc_c, [idx]) + c)
        return _

    lax.fori_loop(0, niter, body, 0)

    def _fold_and_store(acc, slot):
        s = acc[pl.ds(0, 16)]
        for j in range(1, 16):
            s = s + acc[pl.ds(j * 16, 16)]
        fold_v[...] = s
        pltpu.sync_copy(fold_v, out_hbm.at[pl.ds(wid * 48 + slot * 16, 16)])

    _fold_and_store(acc_n, 0)
    _fold_and_store(acc_r, 1)
    _fold_and_store(acc_c, 2)


@functools.cache
def _sc_call():
    mesh = plsc.VectorSubcoreMesh(core_axis_name="c", subcore_axis_name="s")
    return pl.kernel(
        _sc_hist,
        mesh=mesh,
        out_type=jax.ShapeDtypeStruct((_NW * 3 * 16,), jnp.float32),
        scratch_types=[
            pltpu.VMEM((_PLAST,), jnp.float32),
            pltpu.VMEM((_PLAST,), jnp.int32),
            pltpu.VMEM((_PLAST,), jnp.int32),
            pltpu.VMEM((_PLAST,), jnp.float32),
            pltpu.VMEM((256,), jnp.float32),
            pltpu.VMEM((256,), jnp.float32),
            pltpu.VMEM((256,), jnp.float32),
            pltpu.VMEM((16,), jnp.float32),
            pltpu.SemaphoreType.DMA,
        ],
        compiler_params=pltpu.CompilerParams(needs_layout_passes=False),
    )


def kernel(preds, labels):
    labels = labels.astype(jnp.int32)
    conf = _tc_call(preds)
    pflat = preds.reshape(-1)
    parts = _sc_call()(conf, pflat, labels).reshape(_NW, 3, 16)
    tot = jnp.sum(parts, axis=0)
    cnt = tot[0, :_NBINS]
    cor = tot[1, :_NBINS]
    cnf = tot[2, :_NBINS]
    n = jnp.float32(_N)
    safe = jnp.maximum(cnt, 1.0)
    terms = jnp.abs(cnf / safe - cor / safe) * (cnt / n)
    ece = jnp.sum(jnp.where(cnt > 0, terms, 0.0))
    return ece.astype(jnp.float32)
